# baseline (speedup 1.0000x reference)
import jax
import jax.numpy as jnp
from jax import lax
from jax.experimental import pallas as pl
from jax.experimental.pallas import tpu as pltpu

_VOCAB = 100000
_EMBED = 64
_NCTX = 20
_HIDDEN = 256
_BV = 4096
_NB = -(-_VOCAB // _BV)
_OUTW = _NB * _BV
_NEG = -1e30


def _body(x_ref, *refs):
    win_refs = refs[:_NCTX]
    w1_ref, b1_ref, w2t_ref, b2_ref, out_ref, h_ref = refs[_NCTX:]
    i = pl.program_id(0)

    @pl.when(i == 0)
    def _():
        h = b1_ref[...]
        for k in range(_NCTX):
            lane = x_ref[k] % 128
            m = lax.broadcasted_iota(jnp.int32, (_EMBED, 128), 1) == lane
            col = jnp.sum(jnp.where(m, win_refs[k][...], 0.0), axis=1,
                          keepdims=True)
            h = h + lax.dot_general(
                col, w1_ref[k * _EMBED:(k + 1) * _EMBED, :],
                (((0,), (0,)), ((), ())),
                preferred_element_type=jnp.float32)
        h_ref[...] = jnp.maximum(h, 0.0)

    part = lax.dot_general(h_ref[...], w2t_ref[...],
                           (((1,), (1,)), ((), ())),
                           preferred_element_type=jnp.float32)
    col = i * _BV + lax.broadcasted_iota(jnp.int32, (1, _BV), 1)
    logits = jnp.where(col < _VOCAB, part + b2_ref[...], _NEG)
    out_ref[:, pl.ds(i * _BV, _BV)] = logits

    @pl.when(i == _NB - 1)
    def _():
        full = out_ref[...]
        m = jnp.max(full)
        s = jnp.sum(jnp.exp(full - m))
        out_ref[...] = full - (m + jnp.log(s))


_WIN_SPECS = [
    pl.BlockSpec((_EMBED, 128), (lambda i, xr, k=k: (0, xr[k] // 128)))
    for k in range(_NCTX)
]

_GRID_SPEC = pltpu.PrefetchScalarGridSpec(
    num_scalar_prefetch=1,
    grid=(_NB,),
    in_specs=[
        *_WIN_SPECS,
        pl.BlockSpec((_NCTX * _EMBED, _HIDDEN), lambda i, xr: (0, 0)),
        pl.BlockSpec((1, _HIDDEN), lambda i, xr: (0, 0)),
        pl.BlockSpec((_BV, _HIDDEN), lambda i, xr: (i, 0)),
        pl.BlockSpec((1, _BV), lambda i, xr: (0, i)),
    ],
    out_specs=pl.BlockSpec((1, _OUTW), lambda i, xr: (0, 0)),
    scratch_shapes=[pltpu.VMEM((1, _HIDDEN), jnp.float32)],
)


def kernel(x, emb, W1, b1, W2, b2):
    embT = emb.T
    args = [embT] * _NCTX + [W1, b1.reshape(1, _HIDDEN), W2.T,
                             b2.reshape(1, _VOCAB)]
    out = pl.pallas_call(
        _body,
        grid_spec=_GRID_SPEC,
        out_shape=jax.ShapeDtypeStruct((1, _OUTW), jnp.float32),
    )(x.astype(jnp.int32), *args)
    return out[:, :_VOCAB]


# speedup vs baseline: 1.1360x; 1.1360x over previous
import jax
import jax.numpy as jnp
from jax import lax
from jax.experimental import pallas as pl
from jax.experimental.pallas import tpu as pltpu

_VOCAB = 100000
_EMBED = 64
_NCTX = 20
_HIDDEN = 256
_BV = 8192
_NB = -(-_VOCAB // _BV)
_OUTW = _NB * _BV
_NEG = -1e30


def _body(x_ref, *refs):
    win_refs = refs[:_NCTX]
    w1_ref, b1_ref, w2t_ref, b2_ref, out_ref, h_ref, m_ref, s_ref = refs[_NCTX:]
    i = pl.program_id(0)

    @pl.when(i == 0)
    def _():
        h = b1_ref[...]
        for k in range(_NCTX):
            lane = x_ref[k] % 128
            m = lax.broadcasted_iota(jnp.int32, (_EMBED, 128), 1) == lane
            col = jnp.sum(jnp.where(m, win_refs[k][...], 0.0), axis=1,
                          keepdims=True)
            h = h + lax.dot_general(
                col, w1_ref[k * _EMBED:(k + 1) * _EMBED, :],
                (((0,), (0,)), ((), ())),
                preferred_element_type=jnp.float32)
        h_ref[...] = jnp.maximum(h, 0.0)

    part = lax.dot_general(h_ref[...], w2t_ref[...],
                           (((1,), (1,)), ((), ())),
                           preferred_element_type=jnp.float32)
    col = i * _BV + lax.broadcasted_iota(jnp.int32, (1, _BV), 1)
    logits = jnp.where(col < _VOCAB, part + b2_ref[...], _NEG)
    out_ref[:, pl.ds(i * _BV, _BV)] = logits

    bm = jnp.max(logits)

    @pl.when(i == 0)
    def _():
        m_ref[...] = jnp.full((1, 128), bm)
        s_ref[...] = jnp.full((1, 128), jnp.sum(jnp.exp(logits - bm)))

    @pl.when(i > 0)
    def _():
        m_old = jnp.max(m_ref[...])
        m_new = jnp.maximum(m_old, bm)
        s_new = (jnp.max(s_ref[...]) * jnp.exp(m_old - m_new)
                 + jnp.sum(jnp.exp(logits - m_new)))
        m_ref[...] = jnp.full((1, 128), m_new)
        s_ref[...] = jnp.full((1, 128), s_new)

    @pl.when(i == _NB - 1)
    def _():
        lse = jnp.max(m_ref[...]) + jnp.log(jnp.max(s_ref[...]))
        out_ref[...] = out_ref[...] - lse


_WIN_SPECS = [
    pl.BlockSpec((_EMBED, 128), (lambda i, xr, k=k: (0, xr[k] // 128)))
    for k in range(_NCTX)
]

_GRID_SPEC = pltpu.PrefetchScalarGridSpec(
    num_scalar_prefetch=1,
    grid=(_NB,),
    in_specs=[
        *_WIN_SPECS,
        pl.BlockSpec((_NCTX * _EMBED, _HIDDEN), lambda i, xr: (0, 0)),
        pl.BlockSpec((1, _HIDDEN), lambda i, xr: (0, 0)),
        pl.BlockSpec((_BV, _HIDDEN), lambda i, xr: (i, 0)),
        pl.BlockSpec((1, _BV), lambda i, xr: (0, i)),
    ],
    out_specs=pl.BlockSpec((1, _OUTW), lambda i, xr: (0, 0)),
    scratch_shapes=[pltpu.VMEM((1, _HIDDEN), jnp.float32),
                    pltpu.VMEM((1, 128), jnp.float32),
                    pltpu.VMEM((1, 128), jnp.float32)],
)


def kernel(x, emb, W1, b1, W2, b2):
    embT = emb.T
    args = [embT] * _NCTX + [W1, b1.reshape(1, _HIDDEN), W2.T,
                             b2.reshape(1, _VOCAB)]
    out = pl.pallas_call(
        _body,
        grid_spec=_GRID_SPEC,
        out_shape=jax.ShapeDtypeStruct((1, _OUTW), jnp.float32),
    )(x.astype(jnp.int32), *args)
    return out[:, :_VOCAB]


# speedup vs baseline: 1.1602x; 1.0214x over previous
import jax
import jax.numpy as jnp
from jax import lax
from jax.experimental import pallas as pl
from jax.experimental.pallas import tpu as pltpu

_VOCAB = 100000
_EMBED = 64
_NCTX = 20
_HIDDEN = 256
_BV = 8192
_NB = -(-_VOCAB // _BV)
_OUTW = _NB * _BV
_NEG = -1e30


def _body(x_ref, *refs):
    win_refs = refs[:_NCTX]
    w1_ref, b1_ref, w2t_ref, b2_ref, out_ref, h_ref = refs[_NCTX:]
    i = pl.program_id(0)

    @pl.when(i == 0)
    def _():
        h = b1_ref[...]
        for k in range(_NCTX):
            lane = x_ref[k] % 128
            m = lax.broadcasted_iota(jnp.int32, (_EMBED, 128), 1) == lane
            col = jnp.sum(jnp.where(m, win_refs[k][...], 0.0), axis=1,
                          keepdims=True)
            h = h + lax.dot_general(
                col, w1_ref[k * _EMBED:(k + 1) * _EMBED, :],
                (((0,), (0,)), ((), ())),
                preferred_element_type=jnp.float32)
        h_ref[...] = jnp.maximum(h, 0.0)

    part = lax.dot_general(h_ref[...], w2t_ref[...],
                           (((1,), (1,)), ((), ())),
                           preferred_element_type=jnp.float32)
    col = i * _BV + lax.broadcasted_iota(jnp.int32, (1, _BV), 1)
    logits = jnp.where(col < _VOCAB, part + b2_ref[...], _NEG)
    out_ref[:, pl.ds(i * _BV, _BV)] = logits

    @pl.when(i == _NB - 1)
    def _():
        full = out_ref[...]
        m = jnp.max(full)
        s = jnp.sum(jnp.exp(full - m))
        out_ref[...] = full - (m + jnp.log(s))


_WIN_SPECS = [
    pl.BlockSpec((_EMBED, 128), (lambda i, xr, k=k: (0, xr[k] // 128)))
    for k in range(_NCTX)
]

_GRID_SPEC = pltpu.PrefetchScalarGridSpec(
    num_scalar_prefetch=1,
    grid=(_NB,),
    in_specs=[
        *_WIN_SPECS,
        pl.BlockSpec((_NCTX * _EMBED, _HIDDEN), lambda i, xr: (0, 0)),
        pl.BlockSpec((1, _HIDDEN), lambda i, xr: (0, 0)),
        pl.BlockSpec((_BV, _HIDDEN), lambda i, xr: (i, 0)),
        pl.BlockSpec((1, _BV), lambda i, xr: (0, i)),
    ],
    out_specs=pl.BlockSpec((1, _OUTW), lambda i, xr: (0, 0)),
    scratch_shapes=[pltpu.VMEM((1, _HIDDEN), jnp.float32)],
)


def kernel(x, emb, W1, b1, W2, b2):
    embT = emb.T
    args = [embT] * _NCTX + [W1, b1.reshape(1, _HIDDEN), W2.T,
                             b2.reshape(1, _VOCAB)]
    out = pl.pallas_call(
        _body,
        grid_spec=_GRID_SPEC,
        out_shape=jax.ShapeDtypeStruct((1, _OUTW), jnp.float32),
    )(x.astype(jnp.int32), *args)
    return out[:, :_VOCAB]
